# baseline (speedup 1.0000x reference)
import functools

import jax
import jax.numpy as jnp
import numpy as np
from jax import lax
from jax.experimental import pallas as pl
from jax.experimental.pallas import tpu as pltpu
from jax.experimental.pallas import tpu_sc as plsc

N = 10000
E = 320000
D = 128
H = 8
C = 16
ED = 16
NEG = 0.2

NC = 2
NS = 16
NW = NC * NS
EPW = E // NW
G = 80
T = EPW // G
NP = N // 8

_MESH = plsc.VectorSubcoreMesh(
    core_axis_name="c", subcore_axis_name="s", num_cores=NC, num_subcores=NS)


def _wid():
  return lax.axis_index("s") * NC + lax.axis_index("c")


def _make_sc_scatter(NR):
  @functools.partial(
      pl.kernel,
      out_type=jax.ShapeDtypeStruct((NC, NR, D), jnp.float32),
      mesh=_MESH,
      scratch_types=[
          pltpu.VMEM((T, G), jnp.int32),
          pltpu.VMEM((G, D), jnp.float32),
          pltpu.VMEM((G, D), jnp.float32),
          pltpu.VMEM_SHARED((NR, D), jnp.float32),
          pltpu.SemaphoreType.DMA,
      ],
  )
  def k(dst_hbm, rows_hbm, z_hbm, out, idxv, buf0, buf1, acc, sem):
    cid = lax.axis_index("c")
    sid = lax.axis_index("s")
    wid = _wid()

    @pl.when(sid == 0)
    def _init():
      pltpu.sync_copy(z_hbm, acc)

    pltpu.sync_copy(dst_hbm.at[wid], idxv)
    plsc.subcore_barrier()

    def _pair(j0, j1, tail):
      base0 = wid * EPW + j0 * G
      c0 = pltpu.async_copy(rows_hbm.at[pl.ds(base0, G)], buf0, sem)
      if not tail:
        base1 = wid * EPW + j1 * G
        c1 = pltpu.async_copy(rows_hbm.at[pl.ds(base1, G)], buf1, sem)
      c0.wait()
      pltpu.sync_copy(buf0, acc.at[idxv.at[j0]], add=True)
      if not tail:
        c1.wait()
        pltpu.sync_copy(buf1, acc.at[idxv.at[j1]], add=True)

    @pl.loop(0, T - 1, step=2)
    def _chunks(j):
      _pair(j, j + 1, False)

    _pair(T - 1, T - 1, True)

    plsc.subcore_barrier()

    @pl.when(sid == 0)
    def _out():
      pltpu.sync_copy(acc, out.at[cid])

  return k


_sc_sum_n = _make_sc_scatter(N)


@functools.partial(
    pl.kernel,
    out_type=jax.ShapeDtypeStruct((NC, NP, D), jnp.float32),
    mesh=_MESH,
    compiler_params=pltpu.CompilerParams(needs_layout_passes=False),
    scratch_types=[
        pltpu.VMEM((T, G), jnp.int32),
        pltpu.VMEM((T, G), jnp.int32),
        pltpu.VMEM((G // 8, D), jnp.float32),
        pltpu.VMEM((G, D), jnp.float32),
        pltpu.VMEM((G, D), jnp.float32),
        pltpu.VMEM((G, ED), jnp.int32),
        pltpu.VMEM((G, ED), jnp.int32),
        pltpu.VMEM_SHARED((NP, D), jnp.float32),
        pltpu.SemaphoreType.DMA,
    ],
)
def _sc_packed(dstb_hbm, dst8b_hbm, npk_hbm, z_hbm, out,
               idx_d, idx_p, nbuf, pbuf, pbuf1, sbuf, sbuf1, acc, sem):
  cid = lax.axis_index("c")
  sid = lax.axis_index("s")
  wid = _wid()

  @pl.when(sid == 0)
  def _init():
    pltpu.sync_copy(z_hbm, acc)

  pltpu.sync_copy(dstb_hbm.at[wid], idx_d)
  pltpu.sync_copy(dst8b_hbm.at[wid], idx_p)
  pltpu.sync_copy(z_hbm.at[pl.ds(0, G)], pbuf)
  pltpu.sync_copy(z_hbm.at[pl.ds(0, G)], pbuf1)
  plsc.subcore_barrier()

  iota = lax.iota(jnp.int32, 16)
  zeros16 = jnp.zeros((16,), jnp.float32)

  def _build(j, pb, sb):
    pltpu.sync_copy(npk_hbm.at[wid * T + j], nbuf)
    jb = jnp.full((16,), j, jnp.int32)
    for i in range(G):
      ii = jnp.full((16,), i, jnp.int32)
      dv = plsc.load_gather(idx_d, [jb, ii])
      d8 = plsc.load_gather(idx_p, [jb, ii])
      sidx = (dv - d8 * 8) * ED + iota
      payload = plsc.load_gather(
          nbuf, [jnp.full((16,), i // 8, jnp.int32), iota + (i % 8) * ED])
      plsc.store_scatter(pb, [ii, sidx], payload)
      plsc.store_scatter(sb, [ii, iota], sidx)

  def _clean(pb, sb):
    for i in range(G):
      ii = jnp.full((16,), i, jnp.int32)
      sidx = plsc.load_gather(sb, [ii, iota])
      plsc.store_scatter(pb, [ii, sidx], zeros16)

  def _pairp(j0, j1, tail):
    _build(j0, pbuf, sbuf)
    s0 = pltpu.async_copy(pbuf, acc.at[idx_p.at[j0]], sem, add=True)
    if not tail:
      _build(j1, pbuf1, sbuf1)
      s1 = pltpu.async_copy(pbuf1, acc.at[idx_p.at[j1]], sem, add=True)
    s0.wait()
    _clean(pbuf, sbuf)
    if not tail:
      s1.wait()
      _clean(pbuf1, sbuf1)

  @pl.loop(0, T - 1, step=2)
  def _chunks(j):
    _pairp(j, j + 1, False)

  _pairp(T - 1, T - 1, True)

  plsc.subcore_barrier()

  @pl.when(sid == 0)
  def _out():
    pltpu.sync_copy(acc, out.at[cid])


@functools.partial(
    pl.kernel,
    out_type=(jax.ShapeDtypeStruct((E, D), jnp.float32),
              jax.ShapeDtypeStruct((E, D), jnp.float32)),
    mesh=_MESH,
    scratch_types=[
        pltpu.VMEM((T, G), jnp.int32),
        pltpu.VMEM((G, D), jnp.float32),
        pltpu.VMEM((G, D), jnp.float32),
        pltpu.VMEM_SHARED((N, D), jnp.float32),
        pltpu.SemaphoreType.DMA,
        pltpu.SemaphoreType.DMA,
    ],
)
def _sc_gather(src_hbm, dstr_hbm, xl_hbm, xr_hbm, out_l, out_r,
               idx, buf0, buf1, tbl, sem_g, sem_w):
  sid = lax.axis_index("s")
  wid = _wid()

  for idx_hbm, x_hbm, out in ((src_hbm, xl_hbm, out_l),
                              (dstr_hbm, xr_hbm, out_r)):
    @pl.when(sid == 0)
    def _load():
      pltpu.sync_copy(x_hbm, tbl)

    pltpu.sync_copy(idx_hbm.at[wid], idx)
    plsc.subcore_barrier()

    def _pair(j0, j1, tail):
      base0 = wid * EPW + j0 * G
      g0 = pltpu.async_copy(tbl.at[idx.at[j0]], buf0, sem_g)
      if not tail:
        base1 = wid * EPW + j1 * G
        g1 = pltpu.async_copy(tbl.at[idx.at[j1]], buf1, sem_g)
      g0.wait()
      w0 = pltpu.async_copy(buf0, out.at[pl.ds(base0, G)], sem_w)
      if not tail:
        g1.wait()
        w1 = pltpu.async_copy(buf1, out.at[pl.ds(base1, G)], sem_w)
        w1.wait()
      w0.wait()

    @pl.loop(0, T - 1, step=2)
    def _chunks(j):
      _pair(j, j + 1, False)

    _pair(T - 1, T - 1, True)
    plsc.subcore_barrier()


def _tc_proj_body(x_ref, wl_ref, wr_ref, xl_ref, xr_ref):
  xv = x_ref[...]
  xl_ref[...] = jnp.dot(xv, wl_ref[...], preferred_element_type=jnp.float32)
  xr_ref[...] = jnp.dot(xv, wr_ref[...], preferred_element_type=jnp.float32)


def _tc_edge_body(xls_ref, xrd_ref, attr_ref, we_ref, attf_ref,
                  s16_ref, r16_ref, aexp_ref, contrib_ref):
  xls = xls_ref[...]
  e = jnp.dot(attr_ref[...], we_ref[...], preferred_element_type=jnp.float32)
  m = xls + xrd_ref[...] + e
  m = jnp.where(m >= 0, m, NEG * m)
  a16 = jnp.exp(jnp.dot(m * attf_ref[...], s16_ref[...],
                        preferred_element_type=jnp.float32))
  aexp_ref[...] = a16
  contrib_ref[...] = jnp.dot(
      a16, r16_ref[...], preferred_element_type=jnp.float32) * xls


def _tc_ereg_body(attr_ref, we_ref, ereg_ref):
  ereg_ref[...] = jnp.dot(attr_ref[...], we_ref[...],
                          preferred_element_type=jnp.float32)


def _tc_final_body(ap0_ref, ap1_ref, ab0_ref, ab1_ref, es0_ref, es1_ref,
                   xl_ref, xr_ref, x_ref, attf_ref, s16_ref, r16_ref,
                   bias_ref, lns_ref, lnb_ref, out_ref):
  a16 = ap0_ref[...] + ap1_ref[...]
  deg = jnp.maximum(a16[:, 8:9], 1.0)
  el = (es0_ref[...] + es1_ref[...]) / deg
  xl = xl_ref[...]
  m = xl + xr_ref[...] + el
  m = jnp.where(m >= 0, m, NEG * m)
  aloop = jnp.exp(jnp.dot(m * attf_ref[...], s16_ref[...],
                          preferred_element_type=jnp.float32))
  at = a16 + aloop
  denom = jnp.dot(at, r16_ref[...], preferred_element_type=jnp.float32)
  outu = (ab0_ref[...] + ab1_ref[...] +
          jnp.dot(aloop, r16_ref[...],
                  preferred_element_type=jnp.float32) * xl)
  h = outu / (denom + 1e-16) + bias_ref[...] + x_ref[...]
  mu = jnp.mean(h, axis=-1, keepdims=True)
  var = jnp.mean((h - mu) ** 2, axis=-1, keepdims=True)
  h = (h - mu) / jnp.sqrt(var + 1e-5) * lns_ref[...] + lnb_ref[...]
  out_ref[...] = jnp.maximum(h, 0.0)


_B1 = 1000
_B2 = 2000


def _full(shape):
  return pl.BlockSpec(shape, lambda i: tuple(0 for _ in shape))


def _rows(bshape):
  return pl.BlockSpec(bshape, lambda i: (i,) + tuple(0 for _ in bshape[1:]))


def kernel(x, edge_attr, Wl, Wr, We, att, bias, ln_scale, ln_bias, edge_index):
  f32 = jnp.float32
  src = edge_index[0]
  dst = edge_index[1]
  src_r = src.reshape(NW, T, G)
  dst_r = dst.reshape(NW, T, G)
  dst8_r = (dst // 8).reshape(NW, T, G)

  attf = att.reshape(1, H * C)
  s16_np = np.zeros((H * C, ED), np.float32)
  r16_np = np.zeros((ED, H * C), np.float32)
  for h in range(H):
    s16_np[h * C:(h + 1) * C, h] = 1.0
    r16_np[h, h * C:(h + 1) * C] = 1.0
  s16 = jnp.asarray(s16_np)
  r16 = jnp.asarray(r16_np)
  zN = jnp.zeros((N, D), f32)
  zP = jnp.zeros((NP, D), f32)

  ereg = pl.pallas_call(
      _tc_ereg_body,
      grid=(E // _B2,),
      in_specs=[_rows((_B2, ED)), _full((ED, H * C))],
      out_specs=_rows((_B2, D)),
      out_shape=jax.ShapeDtypeStruct((E, D), f32),
  )(edge_attr, We)

  xl, xr = pl.pallas_call(
      _tc_proj_body,
      grid=(N // _B1,),
      in_specs=[_rows((_B1, D)), _full((D, H * C)), _full((D, H * C))],
      out_specs=[_rows((_B1, D)), _rows((_B1, D))],
      out_shape=[
          jax.ShapeDtypeStruct((N, D), f32),
          jax.ShapeDtypeStruct((N, D), f32),
      ],
  )(x, Wl, Wr)

  xls, xrd = _sc_gather(src_r, dst_r, xl, xr)

  aexp, contrib = pl.pallas_call(
      _tc_edge_body,
      grid=(E // _B2,),
      in_specs=[
          _rows((_B2, D)), _rows((_B2, D)), _rows((_B2, ED)),
          _full((ED, H * C)),
          _full((1, H * C)), _full((H * C, ED)), _full((ED, H * C)),
      ],
      out_specs=[_rows((_B2, ED)), _rows((_B2, D))],
      out_shape=[
          jax.ShapeDtypeStruct((E, ED), f32),
          jax.ShapeDtypeStruct((E, D), f32),
      ],
  )(xls, xrd, edge_attr, We, attf, s16, r16)

  acc_es = _sc_sum_n(dst_r, ereg, zN)

  acc_b = _sc_sum_n(dst_r, contrib, zN)
  aexp_pk = aexp.reshape(NW * T, G // 8, 8 * ED)
  acc_p = _sc_packed(dst_r, dst8_r, aexp_pk, zP)
  aexp_sum = acc_p.reshape(NC, N, ED)

  out = pl.pallas_call(
      _tc_final_body,
      grid=(N // _B1,),
      in_specs=[
          _rows((_B1, ED)), _rows((_B1, ED)),
          _rows((_B1, D)), _rows((_B1, D)),
          _rows((_B1, D)), _rows((_B1, D)),
          _rows((_B1, D)), _rows((_B1, D)), _rows((_B1, D)),
          _full((1, H * C)), _full((H * C, ED)), _full((ED, H * C)),
          _full((1, D)), _full((1, D)), _full((1, D)),
      ],
      out_specs=_rows((_B1, D)),
      out_shape=jax.ShapeDtypeStruct((N, D), f32),
  )(aexp_sum[0], aexp_sum[1], acc_b[0], acc_b[1], acc_es[0], acc_es[1],
    xl, xr, x, attf, s16, r16,
    bias.reshape(1, D), ln_scale.reshape(1, D), ln_bias.reshape(1, D))

  return out


# speedup vs baseline: 1.0788x; 1.0788x over previous
import functools

import jax
import jax.numpy as jnp
import numpy as np
from jax import lax
from jax.experimental import pallas as pl
from jax.experimental.pallas import tpu as pltpu
from jax.experimental.pallas import tpu_sc as plsc

N = 10000
E = 320000
D = 128
H = 8
C = 16
ED = 16
NEG = 0.2

NC = 2
NS = 16
NW = NC * NS
EPW = E // NW
G = 80
T = EPW // G
NP = N // 8

_MESH = plsc.VectorSubcoreMesh(
    core_axis_name="c", subcore_axis_name="s", num_cores=NC, num_subcores=NS)


def _wid():
  return lax.axis_index("s") * NC + lax.axis_index("c")


def _make_sc_scatter(NR):
  @functools.partial(
      pl.kernel,
      out_type=jax.ShapeDtypeStruct((NC, NR, D), jnp.float32),
      mesh=_MESH,
      scratch_types=[
          pltpu.VMEM((T, G), jnp.int32),
          pltpu.VMEM((G, D), jnp.float32),
          pltpu.VMEM((G, D), jnp.float32),
          pltpu.VMEM_SHARED((NR, D), jnp.float32),
          pltpu.SemaphoreType.DMA,
      ],
  )
  def k(dst_hbm, rows_hbm, z_hbm, out, idxv, buf0, buf1, acc, sem):
    cid = lax.axis_index("c")
    sid = lax.axis_index("s")
    wid = _wid()

    @pl.when(sid == 0)
    def _init():
      pltpu.sync_copy(z_hbm, acc)

    pltpu.sync_copy(dst_hbm.at[wid], idxv)
    plsc.subcore_barrier()

    def _pair(j0, j1, tail):
      base0 = wid * EPW + j0 * G
      c0 = pltpu.async_copy(rows_hbm.at[pl.ds(base0, G)], buf0, sem)
      if not tail:
        base1 = wid * EPW + j1 * G
        c1 = pltpu.async_copy(rows_hbm.at[pl.ds(base1, G)], buf1, sem)
      c0.wait()
      pltpu.sync_copy(buf0, acc.at[idxv.at[j0]], add=True)
      if not tail:
        c1.wait()
        pltpu.sync_copy(buf1, acc.at[idxv.at[j1]], add=True)

    @pl.loop(0, T - 1, step=2)
    def _chunks(j):
      _pair(j, j + 1, False)

    _pair(T - 1, T - 1, True)

    plsc.subcore_barrier()

    @pl.when(sid == 0)
    def _out():
      pltpu.sync_copy(acc, out.at[cid])

  return k


_sc_sum_n = _make_sc_scatter(N)


@functools.partial(
    pl.kernel,
    out_type=jax.ShapeDtypeStruct((NC, NP, D), jnp.float32),
    mesh=_MESH,
    compiler_params=pltpu.CompilerParams(needs_layout_passes=False),
    scratch_types=[
        pltpu.VMEM((T, G), jnp.int32),
        pltpu.VMEM((T, G), jnp.int32),
        pltpu.VMEM((G // 8, D), jnp.float32),
        pltpu.VMEM((G, D), jnp.float32),
        pltpu.VMEM((G, D), jnp.float32),
        pltpu.VMEM((G, ED), jnp.int32),
        pltpu.VMEM((G, ED), jnp.int32),
        pltpu.VMEM_SHARED((NP, D), jnp.float32),
        pltpu.SemaphoreType.DMA,
    ],
)
def _sc_packed(dstb_hbm, dst8b_hbm, npk_hbm, z_hbm, out,
               idx_d, idx_p, nbuf, pbuf, pbuf1, sbuf, sbuf1, acc, sem):
  cid = lax.axis_index("c")
  sid = lax.axis_index("s")
  wid = _wid()

  @pl.when(sid == 0)
  def _init():
    pltpu.sync_copy(z_hbm, acc)

  pltpu.sync_copy(dstb_hbm.at[wid], idx_d)
  pltpu.sync_copy(dst8b_hbm.at[wid], idx_p)
  pltpu.sync_copy(z_hbm.at[pl.ds(0, G)], pbuf)
  pltpu.sync_copy(z_hbm.at[pl.ds(0, G)], pbuf1)
  plsc.subcore_barrier()

  iota = lax.iota(jnp.int32, 16)
  zeros16 = jnp.zeros((16,), jnp.float32)

  def _build(j, pb, sb):
    pltpu.sync_copy(npk_hbm.at[wid * T + j], nbuf)
    jb = jnp.full((16,), j, jnp.int32)
    for i in range(G):
      ii = jnp.full((16,), i, jnp.int32)
      dv = plsc.load_gather(idx_d, [jb, ii])
      d8 = plsc.load_gather(idx_p, [jb, ii])
      sidx = (dv - d8 * 8) * ED + iota
      payload = plsc.load_gather(
          nbuf, [jnp.full((16,), i // 8, jnp.int32), iota + (i % 8) * ED])
      plsc.store_scatter(pb, [ii, sidx], payload)
      plsc.store_scatter(sb, [ii, iota], sidx)

  def _clean(pb, sb):
    for i in range(G):
      ii = jnp.full((16,), i, jnp.int32)
      sidx = plsc.load_gather(sb, [ii, iota])
      plsc.store_scatter(pb, [ii, sidx], zeros16)

  def _pairp(j0, j1, tail):
    _build(j0, pbuf, sbuf)
    s0 = pltpu.async_copy(pbuf, acc.at[idx_p.at[j0]], sem, add=True)
    if not tail:
      _build(j1, pbuf1, sbuf1)
      s1 = pltpu.async_copy(pbuf1, acc.at[idx_p.at[j1]], sem, add=True)
    s0.wait()
    _clean(pbuf, sbuf)
    if not tail:
      s1.wait()
      _clean(pbuf1, sbuf1)

  @pl.loop(0, T - 1, step=2)
  def _chunks(j):
    _pairp(j, j + 1, False)

  _pairp(T - 1, T - 1, True)

  plsc.subcore_barrier()

  @pl.when(sid == 0)
  def _out():
    pltpu.sync_copy(acc, out.at[cid])


@functools.partial(
    pl.kernel,
    out_type=(jax.ShapeDtypeStruct((E, D), jnp.float32),
              jax.ShapeDtypeStruct((E, D), jnp.float32)),
    mesh=_MESH,
    scratch_types=[
        pltpu.VMEM((T, G), jnp.int32),
        pltpu.VMEM((G, D), jnp.float32),
        pltpu.VMEM((G, D), jnp.float32),
        pltpu.VMEM_SHARED((N, D), jnp.float32),
        pltpu.SemaphoreType.DMA,
        pltpu.SemaphoreType.DMA,
    ],
)
def _sc_gather(src_hbm, dstr_hbm, xl_hbm, xr_hbm, out_l, out_r,
               idx, buf0, buf1, tbl, sem_g, sem_w):
  sid = lax.axis_index("s")
  wid = _wid()

  for idx_hbm, x_hbm, out in ((src_hbm, xl_hbm, out_l),
                              (dstr_hbm, xr_hbm, out_r)):
    @pl.when(sid == 0)
    def _load():
      pltpu.sync_copy(x_hbm, tbl)

    pltpu.sync_copy(idx_hbm.at[wid], idx)
    plsc.subcore_barrier()

    def _pair(j0, j1, tail):
      base0 = wid * EPW + j0 * G
      g0 = pltpu.async_copy(tbl.at[idx.at[j0]], buf0, sem_g)
      if not tail:
        base1 = wid * EPW + j1 * G
        g1 = pltpu.async_copy(tbl.at[idx.at[j1]], buf1, sem_g)
      g0.wait()
      w0 = pltpu.async_copy(buf0, out.at[pl.ds(base0, G)], sem_w)
      if not tail:
        g1.wait()
        w1 = pltpu.async_copy(buf1, out.at[pl.ds(base1, G)], sem_w)
        w1.wait()
      w0.wait()

    @pl.loop(0, T - 1, step=2)
    def _chunks(j):
      _pair(j, j + 1, False)

    _pair(T - 1, T - 1, True)
    plsc.subcore_barrier()


def _tc_proj_body(x_ref, wl_ref, wr_ref, xl_ref, xr_ref):
  xv = x_ref[...]
  xl_ref[...] = jnp.dot(xv, wl_ref[...], preferred_element_type=jnp.float32)
  xr_ref[...] = jnp.dot(xv, wr_ref[...], preferred_element_type=jnp.float32)


def _tc_edge_body(xls_ref, xrd_ref, attr_ref, we_ref, attf_ref,
                  s16_ref, r16_ref, aexp_ref, contrib_ref):
  xls = xls_ref[...]
  e = jnp.dot(attr_ref[...], we_ref[...], preferred_element_type=jnp.float32)
  m = xls + xrd_ref[...] + e
  m = jnp.where(m >= 0, m, NEG * m)
  a16 = jnp.exp(jnp.dot(m * attf_ref[...], s16_ref[...],
                        preferred_element_type=jnp.float32))
  aexp_ref[...] = a16
  contrib_ref[...] = jnp.dot(
      a16, r16_ref[...], preferred_element_type=jnp.float32) * xls


def _tc_final_body(ap0_ref, ap1_ref, ab0_ref, ab1_ref, at0_ref, at1_ref,
                   we_ref, xl_ref, xr_ref, x_ref, attf_ref, s16_ref, r16_ref,
                   bias_ref, lns_ref, lnb_ref, out_ref):
  a16 = ap0_ref[...] + ap1_ref[...]
  deg = jnp.maximum(a16[:, 8:9], 1.0)
  la = (at0_ref[...] + at1_ref[...]) / deg
  el = jnp.dot(la, we_ref[...], preferred_element_type=jnp.float32)
  xl = xl_ref[...]
  m = xl + xr_ref[...] + el
  m = jnp.where(m >= 0, m, NEG * m)
  aloop = jnp.exp(jnp.dot(m * attf_ref[...], s16_ref[...],
                          preferred_element_type=jnp.float32))
  at = a16 + aloop
  denom = jnp.dot(at, r16_ref[...], preferred_element_type=jnp.float32)
  outu = (ab0_ref[...] + ab1_ref[...] +
          jnp.dot(aloop, r16_ref[...],
                  preferred_element_type=jnp.float32) * xl)
  h = outu / (denom + 1e-16) + bias_ref[...] + x_ref[...]
  mu = jnp.mean(h, axis=-1, keepdims=True)
  var = jnp.mean((h - mu) ** 2, axis=-1, keepdims=True)
  h = (h - mu) / jnp.sqrt(var + 1e-5) * lns_ref[...] + lnb_ref[...]
  out_ref[...] = jnp.maximum(h, 0.0)


_B1 = 1000
_B2 = 2000


def _full(shape):
  return pl.BlockSpec(shape, lambda i: tuple(0 for _ in shape))


def _rows(bshape):
  return pl.BlockSpec(bshape, lambda i: (i,) + tuple(0 for _ in bshape[1:]))


def kernel(x, edge_attr, Wl, Wr, We, att, bias, ln_scale, ln_bias, edge_index):
  f32 = jnp.float32
  src = edge_index[0]
  dst = edge_index[1]
  src_r = src.reshape(NW, T, G)
  dst_r = dst.reshape(NW, T, G)
  dst8_r = (dst // 8).reshape(NW, T, G)

  attf = att.reshape(1, H * C)
  s16_np = np.zeros((H * C, ED), np.float32)
  r16_np = np.zeros((ED, H * C), np.float32)
  for h in range(H):
    s16_np[h * C:(h + 1) * C, h] = 1.0
    r16_np[h, h * C:(h + 1) * C] = 1.0
  s16 = jnp.asarray(s16_np)
  r16 = jnp.asarray(r16_np)
  zN = jnp.zeros((N, D), f32)
  zP = jnp.zeros((NP, D), f32)

  attr_pk = edge_attr.reshape(NW * T, G // 8, 8 * ED)
  acc_at = _sc_packed(dst_r, dst8_r, attr_pk, zP)
  attr_sum = acc_at.reshape(NC, N, ED)

  xl, xr = pl.pallas_call(
      _tc_proj_body,
      grid=(N // _B1,),
      in_specs=[_rows((_B1, D)), _full((D, H * C)), _full((D, H * C))],
      out_specs=[_rows((_B1, D)), _rows((_B1, D))],
      out_shape=[
          jax.ShapeDtypeStruct((N, D), f32),
          jax.ShapeDtypeStruct((N, D), f32),
      ],
  )(x, Wl, Wr)

  xls, xrd = _sc_gather(src_r, dst_r, xl, xr)

  aexp, contrib = pl.pallas_call(
      _tc_edge_body,
      grid=(E // _B2,),
      in_specs=[
          _rows((_B2, D)), _rows((_B2, D)), _rows((_B2, ED)),
          _full((ED, H * C)),
          _full((1, H * C)), _full((H * C, ED)), _full((ED, H * C)),
      ],
      out_specs=[_rows((_B2, ED)), _rows((_B2, D))],
      out_shape=[
          jax.ShapeDtypeStruct((E, ED), f32),
          jax.ShapeDtypeStruct((E, D), f32),
      ],
  )(xls, xrd, edge_attr, We, attf, s16, r16)

  acc_b = _sc_sum_n(dst_r, contrib, zN)
  aexp_pk = aexp.reshape(NW * T, G // 8, 8 * ED)
  acc_p = _sc_packed(dst_r, dst8_r, aexp_pk, zP)
  aexp_sum = acc_p.reshape(NC, N, ED)

  out = pl.pallas_call(
      _tc_final_body,
      grid=(N // _B1,),
      in_specs=[
          _rows((_B1, ED)), _rows((_B1, ED)),
          _rows((_B1, D)), _rows((_B1, D)),
          _rows((_B1, ED)), _rows((_B1, ED)),
          _full((ED, H * C)),
          _rows((_B1, D)), _rows((_B1, D)), _rows((_B1, D)),
          _full((1, H * C)), _full((H * C, ED)), _full((ED, H * C)),
          _full((1, D)), _full((1, D)), _full((1, D)),
      ],
      out_specs=_rows((_B1, D)),
      out_shape=jax.ShapeDtypeStruct((N, D), f32),
  )(aexp_sum[0], aexp_sum[1], acc_b[0], acc_b[1], attr_sum[0], attr_sum[1],
    We, xl, xr, x, attf, s16, r16,
    bias.reshape(1, D), ln_scale.reshape(1, D), ln_bias.reshape(1, D))

  return out


# speedup vs baseline: 1.1131x; 1.0318x over previous
import functools

import jax
import jax.numpy as jnp
import numpy as np
from jax import lax
from jax.experimental import pallas as pl
from jax.experimental.pallas import tpu as pltpu
from jax.experimental.pallas import tpu_sc as plsc

N = 10000
E = 320000
D = 128
H = 8
C = 16
ED = 16
NEG = 0.2

NC = 2
NS = 16
NW = NC * NS
EPW = E // NW
G = 80
T = EPW // G
NP = N // 8

_MESH = plsc.VectorSubcoreMesh(
    core_axis_name="c", subcore_axis_name="s", num_cores=NC, num_subcores=NS)


def _wid():
  return lax.axis_index("s") * NC + lax.axis_index("c")


def _make_sc_scatter(NR):
  @functools.partial(
      pl.kernel,
      out_type=jax.ShapeDtypeStruct((NC, NR, D), jnp.float32),
      mesh=_MESH,
      scratch_types=[
          pltpu.VMEM((T, G), jnp.int32),
          pltpu.VMEM((G, D), jnp.float32),
          pltpu.VMEM((G, D), jnp.float32),
          pltpu.VMEM_SHARED((NR, D), jnp.float32),
          pltpu.SemaphoreType.DMA,
      ],
  )
  def k(dst_hbm, rows_hbm, z_hbm, out, idxv, buf0, buf1, acc, sem):
    cid = lax.axis_index("c")
    sid = lax.axis_index("s")
    wid = _wid()

    @pl.when(sid == 0)
    def _init():
      pltpu.sync_copy(z_hbm, acc)

    pltpu.sync_copy(dst_hbm.at[wid], idxv)
    plsc.subcore_barrier()

    def _pair(j0, j1, tail):
      base0 = wid * EPW + j0 * G
      c0 = pltpu.async_copy(rows_hbm.at[pl.ds(base0, G)], buf0, sem)
      if not tail:
        base1 = wid * EPW + j1 * G
        c1 = pltpu.async_copy(rows_hbm.at[pl.ds(base1, G)], buf1, sem)
      c0.wait()
      pltpu.sync_copy(buf0, acc.at[idxv.at[j0]], add=True)
      if not tail:
        c1.wait()
        pltpu.sync_copy(buf1, acc.at[idxv.at[j1]], add=True)

    @pl.loop(0, T - 1, step=2)
    def _chunks(j):
      _pair(j, j + 1, False)

    _pair(T - 1, T - 1, True)

    plsc.subcore_barrier()

    @pl.when(sid == 0)
    def _out():
      pltpu.sync_copy(acc, out.at[cid])

  return k


_sc_sum_n = _make_sc_scatter(N)


@functools.partial(
    pl.kernel,
    out_type=jax.ShapeDtypeStruct((NC, NP, D), jnp.float32),
    mesh=_MESH,
    compiler_params=pltpu.CompilerParams(needs_layout_passes=False),
    scratch_types=[
        pltpu.VMEM((T, G), jnp.int32),
        pltpu.VMEM((T, G), jnp.int32),
        pltpu.VMEM((G // 8, D), jnp.float32),
        pltpu.VMEM((G, D), jnp.float32),
        pltpu.VMEM((G, ED), jnp.int32),
        pltpu.VMEM_SHARED((NP, D), jnp.float32),
    ],
)
def _sc_packed(dstb_hbm, dst8b_hbm, npk_hbm, z_hbm, out,
               idx_d, idx_p, nbuf, pbuf, sbuf, acc):
  cid = lax.axis_index("c")
  sid = lax.axis_index("s")
  wid = _wid()

  @pl.when(sid == 0)
  def _init():
    pltpu.sync_copy(z_hbm, acc)

  pltpu.sync_copy(dstb_hbm.at[wid], idx_d)
  pltpu.sync_copy(dst8b_hbm.at[wid], idx_p)
  pltpu.sync_copy(z_hbm.at[pl.ds(0, G)], pbuf)
  plsc.subcore_barrier()

  iota = lax.iota(jnp.int32, 16)
  zeros16 = jnp.zeros((16,), jnp.float32)

  @pl.loop(0, T)
  def _chunk(j):
    pltpu.sync_copy(npk_hbm.at[wid * T + j], nbuf)
    jb = jnp.full((16,), j, jnp.int32)
    for i in range(G):
      ii = jnp.full((16,), i, jnp.int32)
      dv = plsc.load_gather(idx_d, [jb, ii])
      d8 = plsc.load_gather(idx_p, [jb, ii])
      sidx = (dv - d8 * 8) * ED + iota
      payload = plsc.load_gather(
          nbuf, [jnp.full((16,), i // 8, jnp.int32), iota + (i % 8) * ED])
      plsc.store_scatter(pbuf, [ii, sidx], payload)
      plsc.store_scatter(sbuf, [ii, iota], sidx)
    pltpu.sync_copy(pbuf, acc.at[idx_p.at[j]], add=True)
    for i in range(G):
      ii = jnp.full((16,), i, jnp.int32)
      sidx = plsc.load_gather(sbuf, [ii, iota])
      plsc.store_scatter(pbuf, [ii, sidx], zeros16)

  plsc.subcore_barrier()

  @pl.when(sid == 0)
  def _out():
    pltpu.sync_copy(acc, out.at[cid])


@functools.partial(
    pl.kernel,
    out_type=(jax.ShapeDtypeStruct((E, D), jnp.float32),
              jax.ShapeDtypeStruct((E, D), jnp.float32)),
    mesh=_MESH,
    scratch_types=[
        pltpu.VMEM((T, G), jnp.int32),
        pltpu.VMEM((G, D), jnp.float32),
        pltpu.VMEM((G, D), jnp.float32),
        pltpu.VMEM_SHARED((N, D), jnp.float32),
        pltpu.SemaphoreType.DMA,
        pltpu.SemaphoreType.DMA,
    ],
)
def _sc_gather(src_hbm, dstr_hbm, xl_hbm, xr_hbm, out_l, out_r,
               idx, buf0, buf1, tbl, sem_g, sem_w):
  sid = lax.axis_index("s")
  wid = _wid()

  for idx_hbm, x_hbm, out in ((src_hbm, xl_hbm, out_l),
                              (dstr_hbm, xr_hbm, out_r)):
    @pl.when(sid == 0)
    def _load():
      pltpu.sync_copy(x_hbm, tbl)

    pltpu.sync_copy(idx_hbm.at[wid], idx)
    plsc.subcore_barrier()

    def _pair(j0, j1, tail):
      base0 = wid * EPW + j0 * G
      g0 = pltpu.async_copy(tbl.at[idx.at[j0]], buf0, sem_g)
      if not tail:
        base1 = wid * EPW + j1 * G
        g1 = pltpu.async_copy(tbl.at[idx.at[j1]], buf1, sem_g)
      g0.wait()
      w0 = pltpu.async_copy(buf0, out.at[pl.ds(base0, G)], sem_w)
      if not tail:
        g1.wait()
        w1 = pltpu.async_copy(buf1, out.at[pl.ds(base1, G)], sem_w)
        w1.wait()
      w0.wait()

    @pl.loop(0, T - 1, step=2)
    def _chunks(j):
      _pair(j, j + 1, False)

    _pair(T - 1, T - 1, True)
    plsc.subcore_barrier()


def _tc_proj_body(x_ref, wl_ref, wr_ref, xl_ref, xr_ref):
  xv = x_ref[...]
  xl_ref[...] = jnp.dot(xv, wl_ref[...], preferred_element_type=jnp.float32)
  xr_ref[...] = jnp.dot(xv, wr_ref[...], preferred_element_type=jnp.float32)


def _tc_edge_body(xls_ref, xrd_ref, attr_ref, we_ref, attf_ref,
                  s16_ref, r16_ref, aexp_ref, contrib_ref):
  xls = xls_ref[...]
  e = jnp.dot(attr_ref[...], we_ref[...], preferred_element_type=jnp.float32)
  m = xls + xrd_ref[...] + e
  m = jnp.where(m >= 0, m, NEG * m)
  a16 = jnp.exp(jnp.dot(m * attf_ref[...], s16_ref[...],
                        preferred_element_type=jnp.float32))
  aexp_ref[...] = a16
  contrib_ref[...] = jnp.dot(
      a16, r16_ref[...], preferred_element_type=jnp.float32) * xls


def _tc_final_body(ap0_ref, ap1_ref, ab0_ref, ab1_ref, at0_ref, at1_ref,
                   we_ref, xl_ref, xr_ref, x_ref, attf_ref, s16_ref, r16_ref,
                   bias_ref, lns_ref, lnb_ref, out_ref):
  a16 = ap0_ref[...] + ap1_ref[...]
  deg = jnp.maximum(a16[:, 8:9], 1.0)
  la = (at0_ref[...] + at1_ref[...]) / deg
  el = jnp.dot(la, we_ref[...], preferred_element_type=jnp.float32)
  xl = xl_ref[...]
  m = xl + xr_ref[...] + el
  m = jnp.where(m >= 0, m, NEG * m)
  aloop = jnp.exp(jnp.dot(m * attf_ref[...], s16_ref[...],
                          preferred_element_type=jnp.float32))
  at = a16 + aloop
  denom = jnp.dot(at, r16_ref[...], preferred_element_type=jnp.float32)
  outu = (ab0_ref[...] + ab1_ref[...] +
          jnp.dot(aloop, r16_ref[...],
                  preferred_element_type=jnp.float32) * xl)
  h = outu / (denom + 1e-16) + bias_ref[...] + x_ref[...]
  mu = jnp.mean(h, axis=-1, keepdims=True)
  var = jnp.mean((h - mu) ** 2, axis=-1, keepdims=True)
  h = (h - mu) / jnp.sqrt(var + 1e-5) * lns_ref[...] + lnb_ref[...]
  out_ref[...] = jnp.maximum(h, 0.0)


_B1 = 1000
_B2 = 2000


def _full(shape):
  return pl.BlockSpec(shape, lambda i: tuple(0 for _ in shape))


def _rows(bshape):
  return pl.BlockSpec(bshape, lambda i: (i,) + tuple(0 for _ in bshape[1:]))


def kernel(x, edge_attr, Wl, Wr, We, att, bias, ln_scale, ln_bias, edge_index):
  f32 = jnp.float32
  src = edge_index[0]
  dst = edge_index[1]
  src_r = src.reshape(NW, T, G)
  dst_r = dst.reshape(NW, T, G)
  dst8_r = (dst // 8).reshape(NW, T, G)

  attf = att.reshape(1, H * C)
  s16_np = np.zeros((H * C, ED), np.float32)
  r16_np = np.zeros((ED, H * C), np.float32)
  for h in range(H):
    s16_np[h * C:(h + 1) * C, h] = 1.0
    r16_np[h, h * C:(h + 1) * C] = 1.0
  s16 = jnp.asarray(s16_np)
  r16 = jnp.asarray(r16_np)
  zN = jnp.zeros((N, D), f32)
  zP = jnp.zeros((NP, D), f32)

  attr_pk = edge_attr.reshape(NW * T, G // 8, 8 * ED)
  acc_at = _sc_packed(dst_r, dst8_r, attr_pk, zP)
  attr_sum = acc_at.reshape(NC, N, ED)

  xl, xr = pl.pallas_call(
      _tc_proj_body,
      grid=(N // _B1,),
      in_specs=[_rows((_B1, D)), _full((D, H * C)), _full((D, H * C))],
      out_specs=[_rows((_B1, D)), _rows((_B1, D))],
      out_shape=[
          jax.ShapeDtypeStruct((N, D), f32),
          jax.ShapeDtypeStruct((N, D), f32),
      ],
  )(x, Wl, Wr)

  xls, xrd = _sc_gather(src_r, dst_r, xl, xr)

  aexp, contrib = pl.pallas_call(
      _tc_edge_body,
      grid=(E // _B2,),
      in_specs=[
          _rows((_B2, D)), _rows((_B2, D)), _rows((_B2, ED)),
          _full((ED, H * C)),
          _full((1, H * C)), _full((H * C, ED)), _full((ED, H * C)),
      ],
      out_specs=[_rows((_B2, ED)), _rows((_B2, D))],
      out_shape=[
          jax.ShapeDtypeStruct((E, ED), f32),
          jax.ShapeDtypeStruct((E, D), f32),
      ],
  )(xls, xrd, edge_attr, We, attf, s16, r16)

  acc_b = _sc_sum_n(dst_r, contrib, zN)
  aexp_pk = aexp.reshape(NW * T, G // 8, 8 * ED)
  acc_p = _sc_packed(dst_r, dst8_r, aexp_pk, zP)
  aexp_sum = acc_p.reshape(NC, N, ED)

  out = pl.pallas_call(
      _tc_final_body,
      grid=(N // _B1,),
      in_specs=[
          _rows((_B1, ED)), _rows((_B1, ED)),
          _rows((_B1, D)), _rows((_B1, D)),
          _rows((_B1, ED)), _rows((_B1, ED)),
          _full((ED, H * C)),
          _rows((_B1, D)), _rows((_B1, D)), _rows((_B1, D)),
          _full((1, H * C)), _full((H * C, ED)), _full((ED, H * C)),
          _full((1, D)), _full((1, D)), _full((1, D)),
      ],
      out_specs=_rows((_B1, D)),
      out_shape=jax.ShapeDtypeStruct((N, D), f32),
  )(aexp_sum[0], aexp_sum[1], acc_b[0], acc_b[1], attr_sum[0], attr_sum[1],
    We, xl, xr, x, attf, s16, r16,
    bias.reshape(1, D), ln_scale.reshape(1, D), ln_bias.reshape(1, D))

  return out
